# baseline (speedup 1.0000x reference)
import jax
import jax.numpy as jnp
from jax import lax
from jax.experimental import pallas as pl
from jax.experimental.pallas import tpu as pltpu
from jax.experimental.pallas import tpu_sc as plsc

NC = 2
NS = 16
NW = NC * NS
LANES = 16
CH_ROWS = 16
NBUF = 4


def _sc_body(x_hbm, pe_hbm, out_hbm, pe_b, xb0, xb1, xb2, xb3,
             la, lb, lc, ld, sa, sb, sc, sd):
    nbatch, nrows, d = x_hbm.shape
    rows_w = nrows // NW
    nsub = rows_w // CH_ROWS
    nch = nsub * nbatch
    vpr = d // LANES

    cid = lax.axis_index("c")
    sid = lax.axis_index("s")
    row0 = (sid * NC + cid) * rows_w

    bufs = (xb0, xb1, xb2, xb3)
    ldsems = (la, lb, lc, ld)
    stsems = (sa, sb, sc, sd)

    def issue_load(ch):
        sub, b = divmod(ch, nbatch)
        k = ch % NBUF
        return pltpu.async_copy(
            x_hbm.at[b, pl.ds(row0 + sub * CH_ROWS, CH_ROWS)],
            bufs[k], ldsems[k])

    def issue_store(ch):
        sub, b = divmod(ch, nbatch)
        k = ch % NBUF
        return pltpu.async_copy(
            bufs[k], out_hbm.at[b, pl.ds(row0 + sub * CH_ROWS, CH_ROWS)],
            stsems[k])

    def add_pe(buf):
        def body(i, carry):
            r = i // vpr
            c = (i % vpr) * LANES
            plsc.addupdate(buf.at[r, pl.ds(c, LANES)],
                           pe_b[r, pl.ds(c, LANES)])
            return carry
        lax.fori_loop(0, CH_ROWS * vpr, body, 0, unroll=8)

    loads = [None] * nch
    stores = [None] * nch
    for ch in range(NBUF):
        loads[ch] = issue_load(ch)
    for ch in range(nch):
        sub, b = divmod(ch, nbatch)
        if b == 0:
            pltpu.sync_copy(pe_hbm.at[pl.ds(row0 + sub * CH_ROWS, CH_ROWS)],
                            pe_b)
        loads[ch].wait()
        add_pe(bufs[ch % NBUF])
        stores[ch] = issue_store(ch)
        pv = ch - (NBUF - 2)
        if pv >= 0 and pv + NBUF < nch:
            stores[pv].wait()
            loads[pv + NBUF] = issue_load(pv + NBUF)
    for ch in range(max(0, nch - NBUF), nch):
        stores[ch].wait()


def kernel(x, pe_table):
    B, L, D = x.shape
    mesh = plsc.VectorSubcoreMesh(core_axis_name="c", subcore_axis_name="s")
    f = pl.kernel(
        _sc_body,
        mesh=mesh,
        out_type=jax.ShapeDtypeStruct((B, L, D), x.dtype),
        scratch_types=[
            pltpu.VMEM((CH_ROWS, D), jnp.float32),
            pltpu.VMEM((CH_ROWS, D), jnp.float32),
            pltpu.VMEM((CH_ROWS, D), jnp.float32),
            pltpu.VMEM((CH_ROWS, D), jnp.float32),
            pltpu.VMEM((CH_ROWS, D), jnp.float32),
            pltpu.SemaphoreType.DMA,
            pltpu.SemaphoreType.DMA,
            pltpu.SemaphoreType.DMA,
            pltpu.SemaphoreType.DMA,
            pltpu.SemaphoreType.DMA,
            pltpu.SemaphoreType.DMA,
            pltpu.SemaphoreType.DMA,
            pltpu.SemaphoreType.DMA,
        ],
    )
    return f(x, pe_table)
